# baseline (speedup 1.0000x reference)
import functools

import jax
import jax.numpy as jnp
from jax import lax
from jax.experimental import pallas as pl
from jax.experimental.pallas import tpu as pltpu
from jax.experimental.pallas import tpu_sc as plsc

NC, NS = 2, 16
NW = NC * NS
B, L = 1024, 200
ROW = L * L
BPW = B // NW
VECS = ROW // 16
UNROLL = 20
P0 = 1016
NCELL = 192
DIV_MUL, DIV_SHIFT = 10486, 21


def _build_packed_table(boundaries):
    nb = boundaries.shape[0]
    p = jnp.arange(NCELL, dtype=jnp.int32) + P0
    s = lax.bitcast_convert_type(p << 20, jnp.float32)
    s_next = lax.bitcast_convert_type((p + 1) << 20, jnp.float32)
    dlo = jnp.ceil(s)
    base = jnp.searchsorted(boundaries, dlo, side="left").astype(jnp.int32)
    cand = jnp.minimum(base, nb - 1)
    bcand = boundaries[cand]
    has_thr = (base < nb) & (bcand < s_next)
    thr = jnp.where(has_thr, bcand, 2.0 ** 24).astype(jnp.int32)
    return (thr << 6) | base


@functools.cache
def _make_sc_bias_kernel():
    mesh = plsc.VectorSubcoreMesh(
        core_axis_name="c", subcore_axis_name="s", num_cores=NC)

    @functools.partial(
        pl.kernel,
        out_type=jax.ShapeDtypeStruct((B, ROW), jnp.float32),
        mesh=mesh,
        compiler_params=pltpu.CompilerParams(needs_layout_passes=False),
        scratch_types=[
            pltpu.VMEM((BPW * L,), jnp.int32),
            pltpu.VMEM((BPW * L,), jnp.int32),
            pltpu.VMEM((NCELL,), jnp.int32),
            pltpu.VMEM((64,), jnp.float32),
            pltpu.VMEM((ROW,), jnp.float32),
            pltpu.VMEM((ROW,), jnp.float32),
            pltpu.SemaphoreType.DMA,
            pltpu.SemaphoreType.DMA,
        ],
    )
    def _sc_bias_kernel(tsq_hbm, tsk_hbm, packed_hbm, tb_hbm, out_hbm,
                        tsq_v, tsk_v, packed_v, tb_v, out_v0, out_v1,
                        sem0, sem1):
        wid = lax.axis_index("s") * NC + lax.axis_index("c")
        b0 = wid * BPW
        pltpu.sync_copy(tsq_hbm.at[pl.ds(b0 * L, BPW * L)], tsq_v)
        pltpu.sync_copy(tsk_hbm.at[pl.ds(b0 * L, BPW * L)], tsk_v)
        pltpu.sync_copy(packed_hbm, packed_v)
        pltpu.sync_copy(tb_hbm, tb_v)
        lane = lax.iota(jnp.int32, 16)

        def compute_batch(bl, out_v):
            row16 = jnp.full((16,), bl * L, dtype=jnp.int32)

            def vec_body(it, carry2):
                base = it * (16 * UNROLL)
                ts = [base + u * 16 + lane for u in range(UNROLL)]
                iis = [(t * DIV_MUL) >> DIV_SHIFT for t in ts]
                jjs = [t - ii * L for t, ii in zip(ts, iis)]
                qs = [plsc.load_gather(tsq_v, [row16 + ii]) for ii in iis]
                ks = [plsc.load_gather(tsk_v, [row16 + jj]) for jj in jjs]
                ds = [jnp.maximum(jnp.abs(q - k), 1) for q, k in zip(qs, ks)]
                cells = [(lax.bitcast_convert_type(d.astype(jnp.float32),
                                                   jnp.int32) >> 20) - P0
                         for d in ds]
                pks = [plsc.load_gather(packed_v, [c]) for c in cells]
                buckets = [jnp.where(d > (pk >> 6), (pk & 63) + 1, pk & 63)
                           for d, pk in zip(ds, pks)]
                vals = [plsc.load_gather(tb_v, [b]) for b in buckets]
                for u in range(UNROLL):
                    out_v[pl.ds(base + u * 16, 16)] = vals[u]
                return carry2

            lax.fori_loop(0, VECS // UNROLL, vec_body, 0)

        def pair_body(bp, carry):
            bl = 2 * bp

            @pl.when(bp >= 1)
            def _():
                pltpu.make_async_copy(out_v0, out_hbm.at[b0 + bl - 2],
                                      sem0).wait()

            compute_batch(bl, out_v0)
            pltpu.async_copy(out_v0, out_hbm.at[b0 + bl], sem0)

            @pl.when(bp >= 1)
            def _():
                pltpu.make_async_copy(out_v1, out_hbm.at[b0 + bl - 1],
                                      sem1).wait()

            compute_batch(bl + 1, out_v1)
            pltpu.async_copy(out_v1, out_hbm.at[b0 + bl + 1], sem1)
            return carry

        lax.fori_loop(0, BPW // 2, pair_body, 0)
        pltpu.make_async_copy(out_v0, out_hbm.at[b0 + BPW - 2], sem0).wait()
        pltpu.make_async_copy(out_v1, out_hbm.at[b0 + BPW - 1], sem1).wait()

    return _sc_bias_kernel


def kernel(ts_q, ts_k, time_bias, boundaries):
    assert ts_q.shape == (B, L) and ts_k.shape == (B, L)
    tsq = ts_q.astype(jnp.int32).reshape(B * L)
    tsk = ts_k.astype(jnp.int32).reshape(B * L)
    packed = _build_packed_table(boundaries)
    tb = time_bias[:, 0]
    out = _make_sc_bias_kernel()(tsq, tsk, packed, tb)
    return out.reshape(B, 1, L, L)


# speedup vs baseline: 1.0030x; 1.0030x over previous
import functools

import jax
import jax.numpy as jnp
from jax import lax
from jax.experimental import pallas as pl
from jax.experimental.pallas import tpu as pltpu
from jax.experimental.pallas import tpu_sc as plsc

NC, NS = 2, 16
NW = NC * NS
B, L = 1024, 200
ROW = L * L
BPW = B // NW
VECS = ROW // 16
UNROLL = 25
P0 = 1016
NCELL = 192
DIV_MUL, DIV_SHIFT = 10486, 21


def _build_packed_table(boundaries):
    nb = boundaries.shape[0]
    p = jnp.arange(NCELL, dtype=jnp.int32) + P0
    s = lax.bitcast_convert_type(p << 20, jnp.float32)
    s_next = lax.bitcast_convert_type((p + 1) << 20, jnp.float32)
    dlo = jnp.ceil(s)
    base = jnp.searchsorted(boundaries, dlo, side="left").astype(jnp.int32)
    cand = jnp.minimum(base, nb - 1)
    bcand = boundaries[cand]
    has_thr = (base < nb) & (bcand < s_next)
    thr = jnp.where(has_thr, bcand, 2.0 ** 24).astype(jnp.int32)
    return (thr << 6) | base


@functools.cache
def _make_sc_bias_kernel():
    mesh = plsc.VectorSubcoreMesh(
        core_axis_name="c", subcore_axis_name="s", num_cores=NC)

    @functools.partial(
        pl.kernel,
        out_type=jax.ShapeDtypeStruct((B, ROW), jnp.float32),
        mesh=mesh,
        compiler_params=pltpu.CompilerParams(needs_layout_passes=False),
        scratch_types=[
            pltpu.VMEM((BPW * L,), jnp.int32),
            pltpu.VMEM((BPW * L,), jnp.int32),
            pltpu.VMEM((NCELL,), jnp.int32),
            pltpu.VMEM((64,), jnp.float32),
            pltpu.VMEM((ROW,), jnp.float32),
            pltpu.VMEM((ROW,), jnp.float32),
            pltpu.SemaphoreType.DMA,
            pltpu.SemaphoreType.DMA,
        ],
    )
    def _sc_bias_kernel(tsq_hbm, tsk_hbm, packed_hbm, tb_hbm, out_hbm,
                        tsq_v, tsk_v, packed_v, tb_v, out_v0, out_v1,
                        sem0, sem1):
        wid = lax.axis_index("s") * NC + lax.axis_index("c")
        b0 = wid * BPW
        pltpu.sync_copy(tsq_hbm.at[pl.ds(b0 * L, BPW * L)], tsq_v)
        pltpu.sync_copy(tsk_hbm.at[pl.ds(b0 * L, BPW * L)], tsk_v)
        pltpu.sync_copy(packed_hbm, packed_v)
        pltpu.sync_copy(tb_hbm, tb_v)
        lane = lax.iota(jnp.int32, 16)

        def compute_batch(bl, out_v):
            row16 = jnp.full((16,), bl * L, dtype=jnp.int32)

            def vec_body(it, carry2):
                base = it * (16 * UNROLL)
                ts = [base + u * 16 + lane for u in range(UNROLL)]
                iis = [(t * DIV_MUL) >> DIV_SHIFT for t in ts]
                jjs = [t - ii * L for t, ii in zip(ts, iis)]
                qs = [plsc.load_gather(tsq_v, [row16 + ii]) for ii in iis]
                ks = [plsc.load_gather(tsk_v, [row16 + jj]) for jj in jjs]
                ds = [jnp.maximum(jnp.abs(q - k), 1) for q, k in zip(qs, ks)]
                cells = [(lax.bitcast_convert_type(d.astype(jnp.float32),
                                                   jnp.int32) >> 20) - P0
                         for d in ds]
                pks = [plsc.load_gather(packed_v, [c]) for c in cells]
                buckets = [jnp.where(d > (pk >> 6), (pk & 63) + 1, pk & 63)
                           for d, pk in zip(ds, pks)]
                vals = [plsc.load_gather(tb_v, [b]) for b in buckets]
                for u in range(UNROLL):
                    out_v[pl.ds(base + u * 16, 16)] = vals[u]
                return carry2

            lax.fori_loop(0, VECS // UNROLL, vec_body, 0)

        def pair_body(bp, carry):
            bl = 2 * bp

            @pl.when(bp >= 1)
            def _():
                pltpu.make_async_copy(out_v0, out_hbm.at[b0 + bl - 2],
                                      sem0).wait()

            compute_batch(bl, out_v0)
            pltpu.async_copy(out_v0, out_hbm.at[b0 + bl], sem0)

            @pl.when(bp >= 1)
            def _():
                pltpu.make_async_copy(out_v1, out_hbm.at[b0 + bl - 1],
                                      sem1).wait()

            compute_batch(bl + 1, out_v1)
            pltpu.async_copy(out_v1, out_hbm.at[b0 + bl + 1], sem1)
            return carry

        lax.fori_loop(0, BPW // 2, pair_body, 0)
        pltpu.make_async_copy(out_v0, out_hbm.at[b0 + BPW - 2], sem0).wait()
        pltpu.make_async_copy(out_v1, out_hbm.at[b0 + BPW - 1], sem1).wait()

    return _sc_bias_kernel


def kernel(ts_q, ts_k, time_bias, boundaries):
    assert ts_q.shape == (B, L) and ts_k.shape == (B, L)
    tsq = ts_q.astype(jnp.int32).reshape(B * L)
    tsk = ts_k.astype(jnp.int32).reshape(B * L)
    packed = _build_packed_table(boundaries)
    tb = time_bias[:, 0]
    out = _make_sc_bias_kernel()(tsq, tsk, packed, tb)
    return out.reshape(B, 1, L, L)
